# baseline (speedup 1.0000x reference)
import jax
import jax.numpy as jnp
from jax import lax
from jax.experimental import pallas as pl
from jax.experimental.pallas import tpu as pltpu
from jax.experimental.pallas import tpu_sc as plsc

N = 16384
C = 26
K = 100
W = C * K
NC, NS, L = 2, 16, 16
NW = NC * NS
RPW = N // NW
FPS = 7
CSC = FPS * 2 * K
CTC = W - CSC
CB = 2 * K
RCH = 256
NRC = RPW // RCH
CH = FPS * NRC

BR = 2048


def _onehot_body(xt_hbm, cards_hbm, out_hbm, xv, cards_v, buf0, buf1,
                 sem0, sem1):
    bufs = (buf0, buf1)
    sems = (sem0, sem1)
    wid = lax.axis_index("s") * NC + lax.axis_index("c")
    rbase = pl.multiple_of(wid * RPW, RPW)

    pltpu.sync_copy(xt_hbm.at[:, pl.ds(rbase, RPW)], xv)
    pltpu.sync_copy(cards_hbm, cards_v)

    zeros16 = jnp.zeros((L,), jnp.float32)
    ones16 = jnp.ones((L,), jnp.float32)
    iota16 = lax.iota(jnp.int32, L)

    def zbody(r, _):
        for j in range(RCH // L):
            buf0[r, pl.ds(j * L, L)] = zeros16
            buf1[r, pl.ds(j * L, L)] = zeros16
        return 0
    lax.fori_loop(0, CB, zbody, 0)

    def scatter(ch, buf, val, mask_valid):
        fp = ch // NRC
        r0 = (ch % NRC) * RCH
        for fld in range(2):
            f = fp * 2 + fld
            for g in range(RCH // L):
                vals = xv[f, pl.ds(r0 + g * L, L)]
                rows = vals + fld * K if fld else vals
                cols = g * L + iota16
                if mask_valid:
                    cards_l = cards_v[f, :]
                    plsc.store_scatter(buf, [rows, cols], val,
                                       mask=vals < cards_l)
                else:
                    plsc.store_scatter(buf, [rows, cols], val)

    def start_out(ch, buf, sem):
        fp = ch // NRC
        row0 = pl.multiple_of(fp * CB, 8)
        col0 = pl.multiple_of(rbase + (ch % NRC) * RCH, RCH)
        pltpu.async_copy(buf, out_hbm.at[pl.ds(row0, CB), pl.ds(col0, RCH)],
                         sem)

    def wait_out(buf, sem):
        pltpu.make_async_copy(
            buf, out_hbm.at[pl.ds(0, CB), pl.ds(rbase, RCH)], sem).wait()

    for b in range(2):
        scatter(b, bufs[b], ones16, True)
        start_out(b, bufs[b], sems[b])

    def step(s, _):
        ch0 = 2 + s * 2
        for b in range(2):
            ch = ch0 + b
            wait_out(bufs[b], sems[b])
            scatter(ch - 2, bufs[b], zeros16, False)
            scatter(ch, bufs[b], ones16, True)
            start_out(ch, bufs[b], sems[b])
        return 0
    lax.fori_loop(0, (CH - 2) // 2, step, 0)

    for b in range(2):
        wait_out(bufs[b], sems[b])


def _tc_body(f2_ref, xt2_ref, out_ref):
    delta = jnp.dot(f2_ref[...], xt2_ref[...],
                    preferred_element_type=jnp.float32)
    out_ref[...] = (delta == 0.0).astype(jnp.float32)


@jax.jit
def _onehot(xt, cards_b, f2, xt2):
    mesh = plsc.VectorSubcoreMesh(core_axis_name="c", subcore_axis_name="s")
    sc_call = pl.kernel(
        _onehot_body,
        out_type=jax.ShapeDtypeStruct((CSC, N), jnp.float32),
        mesh=mesh,
        compiler_params=pltpu.CompilerParams(
            needs_layout_passes=False, use_tc_tiling_on_sc=True),
        scratch_types=[
            pltpu.VMEM((C, RPW), jnp.int32),
            pltpu.VMEM((C, L), jnp.int32),
            pltpu.VMEM((CB, RCH), jnp.float32),
            pltpu.VMEM((CB, RCH), jnp.float32),
            pltpu.SemaphoreType.DMA,
            pltpu.SemaphoreType.DMA,
        ],
    )
    out_sc = sc_call(xt, cards_b)
    out_tc = pl.pallas_call(
        _tc_body,
        out_shape=jax.ShapeDtypeStruct((CTC, N), jnp.float32),
        grid=(CTC // CB, N // BR),
        in_specs=[
            pl.BlockSpec((CB, 32), lambda i, j: (i, 0)),
            pl.BlockSpec((32, BR), lambda i, j: (0, j)),
        ],
        out_specs=pl.BlockSpec((CB, BR), lambda i, j: (i, j)),
        compiler_params=pltpu.CompilerParams(
            dimension_semantics=("parallel", "parallel")),
    )(f2, xt2)
    return jnp.concatenate([out_sc, out_tc], axis=0)


def kernel(x, cardinalities):
    xt = x.astype(jnp.int32).T
    cards = jnp.asarray(cardinalities, jnp.int32)
    cards_b = jnp.tile(cards[:, None], (1, L))

    col = jnp.arange(CSC, W, dtype=jnp.int32)
    field = col // K
    cls = (col % K).astype(jnp.float32)
    invalid = (col % K) >= cards[field]
    bias = jnp.where(invalid, cls + 1000.0, cls)
    f2 = jnp.zeros((CTC, 32), jnp.float32)
    f2 = f2.at[jnp.arange(CTC), field].set(1.0)
    f2 = f2.at[:, C].set(-bias)
    xt2 = jnp.zeros((32, N), jnp.float32)
    xt2 = xt2.at[:C, :].set(xt.astype(jnp.float32))
    xt2 = xt2.at[C, :].set(1.0)

    out_t = _onehot(xt, cards_b, f2, xt2)
    return out_t.T


# speedup vs baseline: 2.6225x; 2.6225x over previous
import jax
import jax.numpy as jnp
from jax import lax
from jax.experimental import pallas as pl
from jax.experimental.pallas import tpu as pltpu
from jax.experimental.pallas import tpu_sc as plsc

N = 16384
C = 26
K = 100
W = C * K
NC, NS, L = 2, 16, 16
NW = NC * NS
RPW = N // NW
FP = C // 2
CB = 2 * K
RCH = 256
NRC = RPW // RCH
CH = FP * NRC


def _onehot_body(xt_hbm, cards_hbm, out_hbm, xv, cards_v, buf0, buf1,
                 sem0, sem1):
    bufs = (buf0, buf1)
    sems = (sem0, sem1)
    wid = lax.axis_index("s") * NC + lax.axis_index("c")
    rbase = pl.multiple_of(wid * RPW, RPW)

    pltpu.sync_copy(xt_hbm.at[:, pl.ds(rbase, RPW)], xv)
    pltpu.sync_copy(cards_hbm, cards_v)

    zeros16 = jnp.zeros((L,), jnp.float32)
    ones16 = jnp.ones((L,), jnp.float32)
    iota16 = lax.iota(jnp.int32, L)

    def zbody(r, _):
        for j in range(RCH // L):
            buf0[r, pl.ds(j * L, L)] = zeros16
            buf1[r, pl.ds(j * L, L)] = zeros16
        return 0
    lax.fori_loop(0, CB, zbody, 0)

    def scatter(ch, buf, val, mask_valid):
        fp = ch // NRC
        r0 = (ch % NRC) * RCH
        for fld in range(2):
            f = fp * 2 + fld
            for g in range(RCH // L):
                vals = xv[f, pl.ds(r0 + g * L, L)]
                rows = vals + fld * K if fld else vals
                cols = g * L + iota16
                if mask_valid:
                    cards_l = cards_v[f, :]
                    plsc.store_scatter(buf, [rows, cols], val,
                                       mask=vals < cards_l)
                else:
                    plsc.store_scatter(buf, [rows, cols], val)

    def start_out(ch, buf, sem):
        fp = ch // NRC
        row0 = pl.multiple_of(fp * CB, 8)
        col0 = pl.multiple_of(rbase + (ch % NRC) * RCH, RCH)
        pltpu.async_copy(buf, out_hbm.at[pl.ds(row0, CB), pl.ds(col0, RCH)],
                         sem)

    def wait_out(buf, sem):
        pltpu.make_async_copy(
            buf, out_hbm.at[pl.ds(0, CB), pl.ds(rbase, RCH)], sem).wait()

    for b in range(2):
        scatter(b, bufs[b], ones16, True)
        start_out(b, bufs[b], sems[b])

    def step(s, _):
        ch0 = 2 + s * 2
        for b in range(2):
            ch = ch0 + b
            wait_out(bufs[b], sems[b])
            scatter(ch - 2, bufs[b], zeros16, False)
            scatter(ch, bufs[b], ones16, True)
            start_out(ch, bufs[b], sems[b])
        return 0
    lax.fori_loop(0, (CH - 2) // 2, step, 0)

    for b in range(2):
        wait_out(bufs[b], sems[b])


@jax.jit
def _onehot_sc(xt, cards_b):
    mesh = plsc.VectorSubcoreMesh(core_axis_name="c", subcore_axis_name="s")
    f = pl.kernel(
        _onehot_body,
        out_type=jax.ShapeDtypeStruct((W, N), jnp.float32),
        mesh=mesh,
        compiler_params=pltpu.CompilerParams(
            needs_layout_passes=False, use_tc_tiling_on_sc=True),
        scratch_types=[
            pltpu.VMEM((C, RPW), jnp.int32),
            pltpu.VMEM((C, L), jnp.int32),
            pltpu.VMEM((CB, RCH), jnp.float32),
            pltpu.VMEM((CB, RCH), jnp.float32),
            pltpu.SemaphoreType.DMA,
            pltpu.SemaphoreType.DMA,
        ],
    )
    return f(xt, cards_b)


def kernel(x, cardinalities):
    xt = x.astype(jnp.int32).T
    cards = jnp.asarray(cardinalities, jnp.int32)
    cards_b = jnp.tile(cards[:, None], (1, L))
    out_t = _onehot_sc(xt, cards_b)
    return out_t.T
